# Initial kernel scaffold; baseline (speedup 1.0000x reference)
import jax
import jax.numpy as jnp
from jax.experimental import pallas as pl


def kernel(x, table):
    raise NotImplementedError("write your pallas kernel here")


# speedup vs baseline: 9.0140x; 9.0140x over previous
import functools

import jax
import jax.numpy as jnp
from jax import lax
from jax.experimental import pallas as pl
from jax.experimental.pallas import tpu as pltpu
from jax.experimental.pallas import tpu_sc as plsc

DIM = 128
NC = 2
NS = 16
NW = NC * NS
CH = 128
NBUF = 2


@functools.lru_cache(maxsize=None)
def _build(B, V):
    BPW = B // NW
    NCH = BPW // CH
    G = NCH // NBUF
    mesh = plsc.VectorSubcoreMesh(core_axis_name="c", subcore_axis_name="s")

    @functools.partial(
        pl.kernel,
        mesh=mesh,
        out_type=jax.ShapeDtypeStruct((B, DIM), jnp.float32),
        scratch_types=[
            pltpu.VMEM((NCH, CH), jnp.int32),
            *[pltpu.VMEM((CH, DIM), jnp.float32) for _ in range(NBUF)],
            *[pltpu.SemaphoreType.DMA for _ in range(NBUF)],
        ],
    )
    def emb(idx_hbm, table_hbm, out_hbm, idx_v, *bufs_sems):
        bufs = bufs_sems[:NBUF]
        sems = bufs_sems[NBUF:]
        wid = lax.axis_index("s") * NC + lax.axis_index("c")
        base = wid * BPW
        pltpu.sync_copy(idx_hbm.at[wid], idx_v)

        def gather(j, b):
            return pltpu.make_async_copy(
                table_hbm.at[idx_v.at[j]], bufs[b], sems[b])

        for b in range(NBUF):
            gather(b, b).start()

        def body(g, carry):
            for b in range(NBUF):
                j = g * NBUF + b
                gather(j, b).wait()
                pltpu.sync_copy(bufs[b], out_hbm.at[pl.ds(base + j * CH, CH)])
                gather(j + NBUF, b).start()
            return carry

        lax.fori_loop(0, G - 1, body, 0)

        for b in range(NBUF):
            j = (G - 1) * NBUF + b
            gather(j, b).wait()
            pltpu.sync_copy(bufs[b], out_hbm.at[pl.ds(base + j * CH, CH)])

    return emb


def kernel(x, table):
    S0, S1 = x.shape
    B = S0 * S1
    idx = x.reshape(NW, B // NW // CH, CH).astype(jnp.int32)
    out = _build(B, table.shape[0])(idx, table)
    return out.reshape(S0, S1, DIM)


# speedup vs baseline: 9.1708x; 1.0174x over previous
import functools

import jax
import jax.numpy as jnp
from jax import lax
from jax.experimental import pallas as pl
from jax.experimental.pallas import tpu as pltpu
from jax.experimental.pallas import tpu_sc as plsc

DIM = 128
NC = 2
NS = 16
NW = NC * NS
CH = 128
NBUF = 4


@functools.lru_cache(maxsize=None)
def _build(B, V):
    BPW = B // NW
    NCH = BPW // CH
    G = NCH // NBUF
    mesh = plsc.VectorSubcoreMesh(core_axis_name="c", subcore_axis_name="s")

    @functools.partial(
        pl.kernel,
        mesh=mesh,
        out_type=jax.ShapeDtypeStruct((B, DIM), jnp.float32),
        scratch_types=[
            pltpu.VMEM((NCH, CH), jnp.int32),
            *[pltpu.VMEM((CH, DIM), jnp.float32) for _ in range(NBUF)],
            *[pltpu.SemaphoreType.DMA for _ in range(NBUF)],
        ],
    )
    def emb(idx_hbm, table_hbm, out_hbm, idx_v, *bufs_sems):
        bufs = bufs_sems[:NBUF]
        sems = bufs_sems[NBUF:]
        wid = lax.axis_index("s") * NC + lax.axis_index("c")
        base = wid * BPW
        pltpu.sync_copy(idx_hbm.at[wid], idx_v)

        def gather(j, b):
            return pltpu.make_async_copy(
                table_hbm.at[idx_v.at[j]], bufs[b], sems[b])

        for b in range(NBUF):
            gather(b, b).start()

        def body(g, carry):
            for b in range(NBUF):
                j = g * NBUF + b
                gather(j, b).wait()
                pltpu.sync_copy(bufs[b], out_hbm.at[pl.ds(base + j * CH, CH)])
                gather(j + NBUF, b).start()
            return carry

        lax.fori_loop(0, G - 1, body, 0)

        for b in range(NBUF):
            j = (G - 1) * NBUF + b
            gather(j, b).wait()
            pltpu.sync_copy(bufs[b], out_hbm.at[pl.ds(base + j * CH, CH)])

    return emb


def kernel(x, table):
    S0, S1 = x.shape
    B = S0 * S1
    idx = x.reshape(NW, B // NW // CH, CH).astype(jnp.int32)
    out = _build(B, table.shape[0])(idx, table)
    return out.reshape(S0, S1, DIM)
